# baseline (speedup 1.0000x reference)
import functools

import jax
import jax.numpy as jnp
from jax import lax
from jax.experimental import pallas as pl
from jax.experimental.pallas import tpu as pltpu
from jax.experimental.pallas import tpu_sc as plsc

N = 10000
E = 320000
IN_CH = 128
HID = 16
OUT_CH = 128

NC = 2
NS = 16
NW = NC * NS
L = 16

EPW = E // NW
B = 128
NBUF = 4
C = 80
EPW_P = C * B
K = 4
KB = K * B
SCH = C // K
G = SCH // NBUF
RPT = 632
NPAD = NS * RPT

_MESH = plsc.VectorSubcoreMesh(core_axis_name="c", subcore_axis_name="s")
_SC_PARAMS = pltpu.CompilerParams(use_tc_tiling_on_sc=False)


def _zero_slice_and_barrier(stage_v, acc_sh, s):
    def zfill(i, _):
        stage_v[i, :] = jnp.zeros((L,), jnp.float32)
        return 0
    lax.fori_loop(0, RPT, zfill, 0)
    pltpu.sync_copy(stage_v, acc_sh.at[pl.ds(s * RPT, RPT)])
    plsc.subcore_barrier()


def _writeback(stage_v, acc_sh, out_hbm, c, s):
    plsc.subcore_barrier()
    pltpu.sync_copy(acc_sh.at[pl.ds(s * RPT, RPT)],
                    out_hbm.at[c, pl.ds(s * RPT, RPT)])


@functools.partial(
    pl.kernel,
    out_type=jax.ShapeDtypeStruct((NC, NPAD, L), jnp.float32),
    mesh=_MESH,
    scratch_types=[
        pltpu.VMEM((SCH, KB), jnp.int32),
        pltpu.VMEM((KB, L), jnp.float32),
        pltpu.VMEM((RPT, L), jnp.float32),
        pltpu.VMEM_SHARED((NPAD, L), jnp.float32),
        pltpu.SemaphoreType.DMA,
    ],
    compiler_params=_SC_PARAMS,
)
def _deg_kernel(dst_hbm, out_hbm, dst_v, ones_v, stage_v, acc_sh, sem):
    c = lax.axis_index("c")
    s = lax.axis_index("s")
    w = c * NS + s

    def ofill(i, _):
        ones_v[i, :] = jnp.full((L,), 1.0, jnp.float32)
        return 0
    lax.fori_loop(0, KB, ofill, 0)
    _zero_slice_and_barrier(stage_v, acc_sh, s)

    pltpu.sync_copy(dst_hbm.at[w], dst_v)

    def fire(j, _):
        pltpu.async_copy(ones_v, acc_sh.at[dst_v.at[j]], sem, add=True)
        return 0
    lax.fori_loop(0, SCH, fire, 0)

    def drain(j, _):
        pltpu.make_async_copy(ones_v, acc_sh.at[dst_v.at[j]], sem).wait()
        return 0
    lax.fori_loop(0, SCH, drain, 0)

    _writeback(stage_v, acc_sh, out_hbm, c, s)


@functools.partial(
    pl.kernel,
    out_type=jax.ShapeDtypeStruct((NC, NPAD, L), jnp.float32),
    mesh=_MESH,
    scratch_types=[
        pltpu.VMEM((SCH, KB), jnp.int32),
        pltpu.VMEM((SCH, KB), jnp.int32),
        pltpu.VMEM((NBUF, KB, L), jnp.float32),
        pltpu.VMEM((RPT, L), jnp.float32),
        pltpu.VMEM_SHARED((NPAD, L), jnp.float32),
        pltpu.SemaphoreType.DMA((NBUF,)),
        pltpu.SemaphoreType.DMA((NBUF,)),
    ],
    compiler_params=_SC_PARAMS,
)
def _agg_kernel(g_hbm, src_hbm, dst_hbm, out_hbm,
                src_v, dst_v, rows_v, stage_v, acc_sh, gsem, ssem):
    c = lax.axis_index("c")
    s = lax.axis_index("s")
    w = c * NS + s

    _zero_slice_and_barrier(stage_v, acc_sh, s)

    pltpu.sync_copy(src_hbm.at[w], src_v)
    pltpu.sync_copy(dst_hbm.at[w], dst_v)

    def g_desc(t, b):
        return pltpu.make_async_copy(
            g_hbm.at[src_v.at[t]], rows_v.at[b], gsem.at[b])

    def s_desc(t, b):
        return pltpu.make_async_copy(
            rows_v.at[b], acc_sh.at[dst_v.at[t]], ssem.at[b])

    for b in range(NBUF):
        g_desc(b, b).start()

    def group(i, _):
        for b in range(NBUF):
            t = i * NBUF + b
            g_desc(t, b).wait()
            pltpu.async_copy(rows_v.at[b],
                             acc_sh.at[dst_v.at[t]],
                             ssem.at[b], add=True)
        for b in range(NBUF):
            t = i * NBUF + b
            s_desc(t, b).wait()
            g_desc(t + NBUF, b).start()
        return 0
    lax.fori_loop(0, G - 1, group, 0)

    for b in range(NBUF):
        t = (G - 1) * NBUF + b
        g_desc(t, b).wait()
        pltpu.async_copy(rows_v.at[b],
                         acc_sh.at[dst_v.at[t]],
                         ssem.at[b], add=True)
    for b in range(NBUF):
        s_desc((G - 1) * NBUF + b, b).wait()

    _writeback(stage_v, acc_sh, out_hbm, c, s)


def _tc1_body(degp_ref, x_ref, w1_ref, g1_ref, dinv_ref):
    deg = degp_ref[0] + degp_ref[1] + 1.0
    dinv = lax.rsqrt(deg)
    h = jnp.dot(x_ref[:], w1_ref[:], preferred_element_type=jnp.float32)
    g1_ref[:] = h * dinv
    dinv_ref[:] = dinv


def _tc2_body(accp_ref, g1_ref, dinv_ref, b1_ref, g2_ref):
    dinv = dinv_ref[:]
    z = dinv * (accp_ref[0] + accp_ref[1] + g1_ref[:]) + b1_ref[:]
    g2_ref[:] = jnp.maximum(z, 0.0) * dinv


def _tc3_body(accp_ref, g2_ref, dinv_ref, w2_ref, b2_ref, out_ref):
    z = dinv_ref[:] * (accp_ref[0] + accp_ref[1] + g2_ref[:])
    out_ref[:] = (
        jnp.dot(z, w2_ref[:], preferred_element_type=jnp.float32) + b2_ref[:]
    )


def kernel(x, edge_index, W1, b1, W2, b2):
    src = edge_index[0].reshape(NW, EPW)
    dst = edge_index[1].reshape(NW, EPW)
    src_p = jnp.pad(src, ((0, 0), (0, EPW_P - EPW)),
                    constant_values=N).reshape(NW, SCH, KB)
    dst_p = jnp.pad(dst, ((0, 0), (0, EPW_P - EPW)),
                    constant_values=N).reshape(NW, SCH, KB)
    x_p = jnp.pad(x, ((0, NPAD - N), (0, 0)))
    b1r = b1.reshape(1, HID)
    b2r = b2.reshape(1, OUT_CH)

    deg_parts = _deg_kernel(dst_p)

    g1, dinv16 = pl.pallas_call(
        _tc1_body,
        out_shape=(
            jax.ShapeDtypeStruct((NPAD, HID), jnp.float32),
            jax.ShapeDtypeStruct((NPAD, HID), jnp.float32),
        ),
    )(deg_parts, x_p, W1)

    acc1 = _agg_kernel(g1, src_p, dst_p)

    g2 = pl.pallas_call(
        _tc2_body,
        out_shape=jax.ShapeDtypeStruct((NPAD, HID), jnp.float32),
    )(acc1, g1, dinv16, b1r)

    acc2 = _agg_kernel(g2, src_p, dst_p)

    out = pl.pallas_call(
        _tc3_body,
        out_shape=jax.ShapeDtypeStruct((NPAD, OUT_CH), jnp.float32),
    )(acc2, g2, dinv16, W2, b2r)

    return out[:N]


# speedup vs baseline: 1.3381x; 1.3381x over previous
import functools

import jax
import jax.numpy as jnp
from jax import lax
from jax.experimental import pallas as pl
from jax.experimental.pallas import tpu as pltpu
from jax.experimental.pallas import tpu_sc as plsc

N = 10000
E = 320000
IN_CH = 128
HID = 16
OUT_CH = 128

NC = 2
NS = 16
NW = NC * NS
L = 16

EPW = E // NW
B = 128
NBUF = 4
C = 80
EPW_P = C * B
K = 4
KB = K * B
SCH = C // K
G = SCH // NBUF
RPT = 632
NPAD = NS * RPT

_MESH = plsc.VectorSubcoreMesh(core_axis_name="c", subcore_axis_name="s")
_SC_PARAMS = pltpu.CompilerParams(use_tc_tiling_on_sc=False)


def _zero_slice_and_barrier(stage_v, acc_sh, s):
    def zfill(i, _):
        stage_v[i, :] = jnp.zeros((L,), jnp.float32)
        return 0
    lax.fori_loop(0, RPT, zfill, 0)
    pltpu.sync_copy(stage_v, acc_sh.at[pl.ds(s * RPT, RPT)])
    plsc.subcore_barrier()


def _writeback(stage_v, acc_sh, out_hbm, c, s):
    plsc.subcore_barrier()
    pltpu.sync_copy(acc_sh.at[pl.ds(s * RPT, RPT)],
                    out_hbm.at[c, pl.ds(s * RPT, RPT)])


@functools.partial(
    pl.kernel,
    out_type=jax.ShapeDtypeStruct((NC, NPAD, L), jnp.float32),
    mesh=_MESH,
    scratch_types=[
        pltpu.VMEM((SCH, KB), jnp.int32),
        pltpu.VMEM((KB, L), jnp.float32),
        pltpu.VMEM((RPT, L), jnp.float32),
        pltpu.VMEM_SHARED((NPAD, L), jnp.float32),
        pltpu.SemaphoreType.DMA,
    ],
    compiler_params=_SC_PARAMS,
)
def _deg_kernel(dst_hbm, out_hbm, dst_v, ones_v, stage_v, acc_sh, sem):
    c = lax.axis_index("c")
    s = lax.axis_index("s")
    w = c * NS + s

    def ofill(i, _):
        ones_v[i, :] = jnp.full((L,), 1.0, jnp.float32)
        return 0
    lax.fori_loop(0, KB, ofill, 0)
    _zero_slice_and_barrier(stage_v, acc_sh, s)

    pltpu.sync_copy(dst_hbm.at[w], dst_v)

    def fire(j, _):
        pltpu.async_copy(ones_v, acc_sh.at[dst_v.at[j]], sem, add=True)
        return 0
    lax.fori_loop(0, SCH, fire, 0)

    def drain(j, _):
        pltpu.make_async_copy(ones_v, acc_sh.at[dst_v.at[j]], sem).wait()
        return 0
    lax.fori_loop(0, SCH, drain, 0)

    _writeback(stage_v, acc_sh, out_hbm, c, s)


@functools.partial(
    pl.kernel,
    out_type=jax.ShapeDtypeStruct((NC, NPAD, L), jnp.float32),
    mesh=_MESH,
    scratch_types=[
        pltpu.VMEM((SCH, KB), jnp.int32),
        pltpu.VMEM((SCH, KB), jnp.int32),
        pltpu.VMEM((NBUF, KB, L), jnp.float32),
        pltpu.VMEM((RPT, L), jnp.float32),
        pltpu.VMEM_SHARED((NPAD, L), jnp.float32),
        pltpu.VMEM_SHARED((NPAD, L), jnp.float32),
        pltpu.SemaphoreType.DMA((NBUF,)),
        pltpu.SemaphoreType.DMA((NBUF,)),
    ],
    compiler_params=_SC_PARAMS,
)
def _agg_kernel(g_hbm, src_hbm, dst_hbm, out_hbm,
                src_v, dst_v, rows_v, stage_v, acc_sh, g_sp, gsem, ssem):
    c = lax.axis_index("c")
    s = lax.axis_index("s")
    w = c * NS + s

    pltpu.sync_copy(g_hbm.at[pl.ds(s * RPT, RPT)],
                    g_sp.at[pl.ds(s * RPT, RPT)])
    _zero_slice_and_barrier(stage_v, acc_sh, s)

    pltpu.sync_copy(src_hbm.at[w], src_v)
    pltpu.sync_copy(dst_hbm.at[w], dst_v)

    def g_desc(t, b):
        return pltpu.make_async_copy(
            g_sp.at[src_v.at[t]], rows_v.at[b], gsem.at[b])

    def s_desc(t, b):
        return pltpu.make_async_copy(
            rows_v.at[b], acc_sh.at[dst_v.at[t]], ssem.at[b])

    for b in range(NBUF):
        g_desc(b, b).start()

    def group(i, _):
        for b in range(NBUF):
            t = i * NBUF + b
            g_desc(t, b).wait()
            pltpu.async_copy(rows_v.at[b],
                             acc_sh.at[dst_v.at[t]],
                             ssem.at[b], add=True)
        for b in range(NBUF):
            t = i * NBUF + b
            s_desc(t, b).wait()
            g_desc(t + NBUF, b).start()
        return 0
    lax.fori_loop(0, G - 1, group, 0)

    for b in range(NBUF):
        t = (G - 1) * NBUF + b
        g_desc(t, b).wait()
        pltpu.async_copy(rows_v.at[b],
                         acc_sh.at[dst_v.at[t]],
                         ssem.at[b], add=True)
    for b in range(NBUF):
        s_desc((G - 1) * NBUF + b, b).wait()

    _writeback(stage_v, acc_sh, out_hbm, c, s)


def _tc1_body(degp_ref, x_ref, w1_ref, g1_ref, dinv_ref):
    deg = degp_ref[0] + degp_ref[1] + 1.0
    dinv = lax.rsqrt(deg)
    h = jnp.dot(x_ref[:], w1_ref[:], preferred_element_type=jnp.float32)
    g1_ref[:] = h * dinv
    dinv_ref[:] = dinv


def _tc2_body(accp_ref, g1_ref, dinv_ref, b1_ref, g2_ref):
    dinv = dinv_ref[:]
    z = dinv * (accp_ref[0] + accp_ref[1] + g1_ref[:]) + b1_ref[:]
    g2_ref[:] = jnp.maximum(z, 0.0) * dinv


def _tc3_body(accp_ref, g2_ref, dinv_ref, w2_ref, b2_ref, out_ref):
    z = dinv_ref[:] * (accp_ref[0] + accp_ref[1] + g2_ref[:])
    out_ref[:] = (
        jnp.dot(z, w2_ref[:], preferred_element_type=jnp.float32) + b2_ref[:]
    )


def kernel(x, edge_index, W1, b1, W2, b2):
    src = edge_index[0].reshape(NW, EPW)
    dst = edge_index[1].reshape(NW, EPW)
    src_p = jnp.pad(src, ((0, 0), (0, EPW_P - EPW)),
                    constant_values=N).reshape(NW, SCH, KB)
    dst_p = jnp.pad(dst, ((0, 0), (0, EPW_P - EPW)),
                    constant_values=N).reshape(NW, SCH, KB)
    x_p = jnp.pad(x, ((0, NPAD - N), (0, 0)))
    b1r = b1.reshape(1, HID)
    b2r = b2.reshape(1, OUT_CH)

    deg_parts = _deg_kernel(dst_p)

    g1, dinv16 = pl.pallas_call(
        _tc1_body,
        out_shape=(
            jax.ShapeDtypeStruct((NPAD, HID), jnp.float32),
            jax.ShapeDtypeStruct((NPAD, HID), jnp.float32),
        ),
    )(deg_parts, x_p, W1)

    acc1 = _agg_kernel(g1, src_p, dst_p)

    g2 = pl.pallas_call(
        _tc2_body,
        out_shape=jax.ShapeDtypeStruct((NPAD, HID), jnp.float32),
    )(acc1, g1, dinv16, b1r)

    acc2 = _agg_kernel(g2, src_p, dst_p)

    out = pl.pallas_call(
        _tc3_body,
        out_shape=jax.ShapeDtypeStruct((NPAD, OUT_CH), jnp.float32),
    )(acc2, g2, dinv16, W2, b2r)

    return out[:N]


# speedup vs baseline: 1.4807x; 1.1066x over previous
import functools

import jax
import jax.numpy as jnp
from jax import lax
from jax.experimental import pallas as pl
from jax.experimental.pallas import tpu as pltpu
from jax.experimental.pallas import tpu_sc as plsc

N = 10000
E = 320000
IN_CH = 128
HID = 16
OUT_CH = 128

NC = 2
NS = 16
NW = NC * NS
L = 16

EPW = E // NW
NBUF = 4
KB = 512
SCH = 20
EPW_P = SCH * KB
G = SCH // NBUF
RPT = 632
NPAD = NS * RPT

_MESH = plsc.VectorSubcoreMesh(core_axis_name="c", subcore_axis_name="s")
_SC_PARAMS = pltpu.CompilerParams(use_tc_tiling_on_sc=False)


def _fill_tail(idx_v):
    for i in range((EPW_P - EPW) // L):
        idx_v[pl.ds(EPW + i * L, L)] = jnp.full((L,), N, jnp.int32)


def _zero_slice_and_barrier(stage_v, acc_sh, s):
    def zfill(i, _):
        stage_v[i, :] = jnp.zeros((L,), jnp.float32)
        return 0
    lax.fori_loop(0, RPT, zfill, 0)
    pltpu.sync_copy(stage_v, acc_sh.at[pl.ds(s * RPT, RPT)])
    plsc.subcore_barrier()


def _writeback(stage_v, acc_sh, out_hbm, c, s):
    plsc.subcore_barrier()
    pltpu.sync_copy(acc_sh.at[pl.ds(s * RPT, RPT)],
                    out_hbm.at[c, pl.ds(s * RPT, RPT)])


@functools.partial(
    pl.kernel,
    out_type=jax.ShapeDtypeStruct((NC, NPAD, L), jnp.float32),
    mesh=_MESH,
    scratch_types=[
        pltpu.VMEM((EPW_P,), jnp.int32),
        pltpu.VMEM((KB, L), jnp.float32),
        pltpu.VMEM((RPT, L), jnp.float32),
        pltpu.VMEM_SHARED((NPAD, L), jnp.float32),
        pltpu.SemaphoreType.DMA,
    ],
    compiler_params=_SC_PARAMS,
)
def _deg_kernel(ei_hbm, out_hbm, dst_v, ones_v, stage_v, acc_sh, sem):
    c = lax.axis_index("c")
    s = lax.axis_index("s")
    w = c * NS + s

    def ofill(i, _):
        ones_v[i, :] = jnp.full((L,), 1.0, jnp.float32)
        return 0
    lax.fori_loop(0, KB, ofill, 0)
    _zero_slice_and_barrier(stage_v, acc_sh, s)

    pltpu.sync_copy(ei_hbm.at[1, pl.ds(w * EPW, EPW)],
                    dst_v.at[pl.ds(0, EPW)])
    _fill_tail(dst_v)

    def fire(j, _):
        pltpu.async_copy(ones_v, acc_sh.at[dst_v.at[pl.ds(j * KB, KB)]],
                         sem, add=True)
        return 0
    lax.fori_loop(0, SCH, fire, 0)

    def drain(j, _):
        pltpu.make_async_copy(ones_v,
                              acc_sh.at[dst_v.at[pl.ds(j * KB, KB)]],
                              sem).wait()
        return 0
    lax.fori_loop(0, SCH, drain, 0)

    _writeback(stage_v, acc_sh, out_hbm, c, s)


@functools.partial(
    pl.kernel,
    out_type=jax.ShapeDtypeStruct((NC, NPAD, L), jnp.float32),
    mesh=_MESH,
    scratch_types=[
        pltpu.VMEM((EPW_P,), jnp.int32),
        pltpu.VMEM((EPW_P,), jnp.int32),
        pltpu.VMEM((NBUF, KB, L), jnp.float32),
        pltpu.VMEM((RPT, L), jnp.float32),
        pltpu.VMEM_SHARED((NPAD, L), jnp.float32),
        pltpu.VMEM_SHARED((NPAD, L), jnp.float32),
        pltpu.SemaphoreType.DMA((NBUF,)),
        pltpu.SemaphoreType.DMA((NBUF,)),
    ],
    compiler_params=_SC_PARAMS,
)
def _agg_kernel(g_hbm, ei_hbm, out_hbm,
                src_v, dst_v, rows_v, stage_v, acc_sh, g_sp, gsem, ssem):
    c = lax.axis_index("c")
    s = lax.axis_index("s")
    w = c * NS + s

    pltpu.sync_copy(g_hbm.at[pl.ds(s * RPT, RPT)],
                    g_sp.at[pl.ds(s * RPT, RPT)])
    _zero_slice_and_barrier(stage_v, acc_sh, s)

    pltpu.sync_copy(ei_hbm.at[0, pl.ds(w * EPW, EPW)],
                    src_v.at[pl.ds(0, EPW)])
    pltpu.sync_copy(ei_hbm.at[1, pl.ds(w * EPW, EPW)],
                    dst_v.at[pl.ds(0, EPW)])
    _fill_tail(src_v)
    _fill_tail(dst_v)

    def g_desc(t, b):
        return pltpu.make_async_copy(
            g_sp.at[src_v.at[pl.ds(t * KB, KB)]], rows_v.at[b], gsem.at[b])

    def s_desc(t, b):
        return pltpu.make_async_copy(
            rows_v.at[b], acc_sh.at[dst_v.at[pl.ds(t * KB, KB)]], ssem.at[b])

    for b in range(NBUF):
        g_desc(b, b).start()

    def group(i, _):
        for b in range(NBUF):
            t = i * NBUF + b
            g_desc(t, b).wait()
            pltpu.async_copy(rows_v.at[b],
                             acc_sh.at[dst_v.at[pl.ds(t * KB, KB)]],
                             ssem.at[b], add=True)
        for b in range(NBUF):
            t = i * NBUF + b
            s_desc(t, b).wait()
            g_desc(t + NBUF, b).start()
        return 0
    lax.fori_loop(0, G - 1, group, 0)

    for b in range(NBUF):
        t = (G - 1) * NBUF + b
        g_desc(t, b).wait()
        pltpu.async_copy(rows_v.at[b],
                         acc_sh.at[dst_v.at[pl.ds(t * KB, KB)]],
                         ssem.at[b], add=True)
    for b in range(NBUF):
        s_desc((G - 1) * NBUF + b, b).wait()

    _writeback(stage_v, acc_sh, out_hbm, c, s)


def _tc1_body(degp_ref, x_ref, w1_ref, g1_ref, dinv_ref):
    deg = degp_ref[0] + degp_ref[1] + 1.0
    dinv = lax.rsqrt(deg)
    h = jnp.dot(x_ref[:], w1_ref[:], preferred_element_type=jnp.float32)
    g1_ref[:N, :] = h * dinv[:N, :]
    g1_ref[N:, :] = jnp.zeros((NPAD - N, HID), jnp.float32)
    dinv_ref[:] = dinv


def _tc2_body(accp_ref, g1_ref, dinv_ref, b1_ref, g2_ref):
    dinv = dinv_ref[:]
    z = dinv * (accp_ref[0] + accp_ref[1] + g1_ref[:]) + b1_ref[:]
    g2_ref[:] = jnp.maximum(z, 0.0) * dinv


def _tc3_body(accp_ref, g2_ref, dinv_ref, w2_ref, b2_ref, out_ref):
    z = dinv_ref[:N, :] * (
        accp_ref[0, :N, :] + accp_ref[1, :N, :] + g2_ref[:N, :])
    out_ref[:] = (
        jnp.dot(z, w2_ref[:], preferred_element_type=jnp.float32) + b2_ref[:]
    )


def kernel(x, edge_index, W1, b1, W2, b2):
    b1r = b1.reshape(1, HID)
    b2r = b2.reshape(1, OUT_CH)

    deg_parts = _deg_kernel(edge_index)

    g1, dinv16 = pl.pallas_call(
        _tc1_body,
        out_shape=(
            jax.ShapeDtypeStruct((NPAD, HID), jnp.float32),
            jax.ShapeDtypeStruct((NPAD, HID), jnp.float32),
        ),
    )(deg_parts, x, W1)

    acc1 = _agg_kernel(g1, edge_index)

    g2 = pl.pallas_call(
        _tc2_body,
        out_shape=jax.ShapeDtypeStruct((NPAD, HID), jnp.float32),
    )(acc1, g1, dinv16, b1r)

    acc2 = _agg_kernel(g2, edge_index)

    out = pl.pallas_call(
        _tc3_body,
        out_shape=jax.ShapeDtypeStruct((N, OUT_CH), jnp.float32),
    )(acc2, g2, dinv16, W2, b2r)

    return out


# speedup vs baseline: 1.5653x; 1.0571x over previous
import functools

import jax
import jax.numpy as jnp
from jax import lax
from jax.experimental import pallas as pl
from jax.experimental.pallas import tpu as pltpu
from jax.experimental.pallas import tpu_sc as plsc

N = 10000
E = 320000
IN_CH = 128
HID = 16
OUT_CH = 128

NC = 2
NS = 16
NW = NC * NS
L = 16

EPW = E // NW
NBUF = 4
KB = 512
SCH = 20
EPW_P = SCH * KB
G = SCH // NBUF
RPT = 632
NPAD = NS * RPT

_MESH = plsc.VectorSubcoreMesh(core_axis_name="c", subcore_axis_name="s")
_SC_PARAMS = pltpu.CompilerParams(use_tc_tiling_on_sc=False)


def _fill_tail(idx_v):
    for i in range((EPW_P - EPW) // L):
        idx_v[pl.ds(EPW + i * L, L)] = jnp.full((L,), N, jnp.int32)


def _zero_slice_and_barrier(stage_v, acc_sh, s):
    def zfill(i, _):
        stage_v[i, :] = jnp.zeros((L,), jnp.float32)
        return 0
    lax.fori_loop(0, RPT, zfill, 0)
    pltpu.sync_copy(stage_v, acc_sh.at[pl.ds(s * RPT, RPT)])
    plsc.subcore_barrier()


def _writeback(stage_v, acc_sh, out_hbm, c, s):
    plsc.subcore_barrier()
    pltpu.sync_copy(acc_sh.at[pl.ds(s * RPT, RPT)],
                    out_hbm.at[c, pl.ds(s * RPT, RPT)])


@functools.partial(
    pl.kernel,
    out_type=jax.ShapeDtypeStruct((NC, NPAD, L), jnp.float32),
    mesh=_MESH,
    scratch_types=[
        pltpu.VMEM((EPW_P,), jnp.int32),
        pltpu.VMEM((KB, L), jnp.float32),
        pltpu.VMEM((RPT, L), jnp.float32),
        pltpu.VMEM_SHARED((NPAD, L), jnp.float32),
        pltpu.SemaphoreType.DMA,
    ],
    compiler_params=_SC_PARAMS,
)
def _deg_kernel(ei_hbm, out_hbm, dst_v, ones_v, stage_v, acc_sh, sem):
    c = lax.axis_index("c")
    s = lax.axis_index("s")
    w = c * NS + s

    def ofill(i, _):
        ones_v[i, :] = jnp.full((L,), 1.0, jnp.float32)
        return 0
    lax.fori_loop(0, KB, ofill, 0)
    _zero_slice_and_barrier(stage_v, acc_sh, s)

    pltpu.sync_copy(ei_hbm.at[1, pl.ds(w * EPW, EPW)],
                    dst_v.at[pl.ds(0, EPW)])
    _fill_tail(dst_v)

    def fire(j, _):
        pltpu.async_copy(ones_v, acc_sh.at[dst_v.at[pl.ds(j * KB, KB)]],
                         sem, add=True)
        return 0
    lax.fori_loop(0, SCH, fire, 0)

    def drain(j, _):
        pltpu.make_async_copy(ones_v,
                              acc_sh.at[dst_v.at[pl.ds(j * KB, KB)]],
                              sem).wait()
        return 0
    lax.fori_loop(0, SCH, drain, 0)

    _writeback(stage_v, acc_sh, out_hbm, c, s)


@functools.partial(
    pl.kernel,
    out_type=jax.ShapeDtypeStruct((NC, NPAD, L), jnp.float32),
    mesh=_MESH,
    scratch_types=[
        pltpu.VMEM((EPW_P,), jnp.int32),
        pltpu.VMEM((EPW_P,), jnp.int32),
        pltpu.VMEM((NBUF, KB, L), jnp.float32),
        pltpu.VMEM((RPT, L), jnp.float32),
        pltpu.VMEM_SHARED((NPAD, L), jnp.float32),
        pltpu.VMEM_SHARED((NPAD, L), jnp.float32),
        pltpu.SemaphoreType.DMA((NBUF,)),
        pltpu.SemaphoreType.DMA((NBUF,)),
    ],
    compiler_params=_SC_PARAMS,
)
def _agg_kernel(g_hbm, ei_hbm, out_hbm,
                src_v, dst_v, rows_v, stage_v, acc_sh, g_sp, gsem, ssem):
    c = lax.axis_index("c")
    s = lax.axis_index("s")
    w = c * NS + s

    pltpu.sync_copy(g_hbm.at[pl.ds(s * RPT, RPT)],
                    g_sp.at[pl.ds(s * RPT, RPT)])
    _zero_slice_and_barrier(stage_v, acc_sh, s)

    pltpu.sync_copy(ei_hbm.at[0, pl.ds(w * EPW, EPW)],
                    src_v.at[pl.ds(0, EPW)])
    pltpu.sync_copy(ei_hbm.at[1, pl.ds(w * EPW, EPW)],
                    dst_v.at[pl.ds(0, EPW)])
    _fill_tail(src_v)
    _fill_tail(dst_v)

    def g_desc(t, b):
        return pltpu.make_async_copy(
            g_sp.at[src_v.at[pl.ds(t * KB, KB)]], rows_v.at[b], gsem.at[b])

    def s_desc(t, b):
        return pltpu.make_async_copy(
            rows_v.at[b], acc_sh.at[dst_v.at[pl.ds(t * KB, KB)]], ssem.at[b])

    for b in range(NBUF):
        g_desc(b, b).start()

    def group(i, _):
        for b in range(NBUF):
            t = i * NBUF + b
            g_desc(t, b).wait()
            pltpu.async_copy(rows_v.at[b],
                             acc_sh.at[dst_v.at[pl.ds(t * KB, KB)]],
                             ssem.at[b], add=True)
        for b in range(NBUF):
            t = i * NBUF + b
            s_desc(t, b).wait()
            g_desc(t + NBUF, b).start()
        return 0
    lax.fori_loop(0, G - 1, group, 0)

    for b in range(NBUF):
        t = (G - 1) * NBUF + b
        g_desc(t, b).wait()
        pltpu.async_copy(rows_v.at[b],
                         acc_sh.at[dst_v.at[pl.ds(t * KB, KB)]],
                         ssem.at[b], add=True)
    for b in range(NBUF):
        s_desc((G - 1) * NBUF + b, b).wait()

    _writeback(stage_v, acc_sh, out_hbm, c, s)


@functools.partial(
    pl.kernel,
    out_type=jax.ShapeDtypeStruct((NC, NPAD, L), jnp.float32),
    mesh=_MESH,
    scratch_types=[
        pltpu.VMEM((EPW_P,), jnp.int32),
        pltpu.VMEM((EPW_P,), jnp.int32),
        pltpu.VMEM((NBUF, KB, L), jnp.float32),
        pltpu.VMEM((RPT, L), jnp.float32),
        pltpu.VMEM((RPT, L), jnp.float32),
        pltpu.VMEM((RPT, L), jnp.float32),
        pltpu.VMEM((RPT, L), jnp.float32),
        pltpu.VMEM((1, L), jnp.float32),
        pltpu.VMEM_SHARED((NPAD, L), jnp.float32),
        pltpu.VMEM_SHARED((NPAD, L), jnp.float32),
        pltpu.SemaphoreType.DMA((NBUF,)),
        pltpu.SemaphoreType.DMA((NBUF,)),
    ],
    compiler_params=_SC_PARAMS,
)
def _agg2_kernel(accp_hbm, g1_hbm, dinv_hbm, b1_hbm, ei_hbm, out_hbm,
                 src_v, dst_v, rows_v, stage_v, tmp_v, dinv_v, g2_v, b1_v,
                 acc_sh, g_sp, gsem, ssem):
    c = lax.axis_index("c")
    s = lax.axis_index("s")
    w = c * NS + s
    sl = pl.ds(s * RPT, RPT)

    pltpu.sync_copy(accp_hbm.at[0, sl], stage_v)
    pltpu.sync_copy(accp_hbm.at[1, sl], tmp_v)
    pltpu.sync_copy(g1_hbm.at[sl], g2_v)
    pltpu.sync_copy(dinv_hbm.at[sl], dinv_v)
    pltpu.sync_copy(b1_hbm, b1_v)

    def prow(i, _):
        acc = stage_v[i, :] + tmp_v[i, :] + g2_v[i, :]
        z = dinv_v[i, :] * acc + b1_v[0, :]
        g2_v[i, :] = jnp.maximum(z, 0.0) * dinv_v[i, :]
        return 0
    lax.fori_loop(0, RPT, prow, 0)
    pltpu.sync_copy(g2_v, g_sp.at[sl])
    _zero_slice_and_barrier(stage_v, acc_sh, s)

    pltpu.sync_copy(ei_hbm.at[0, pl.ds(w * EPW, EPW)],
                    src_v.at[pl.ds(0, EPW)])
    pltpu.sync_copy(ei_hbm.at[1, pl.ds(w * EPW, EPW)],
                    dst_v.at[pl.ds(0, EPW)])
    _fill_tail(src_v)
    _fill_tail(dst_v)

    def g_desc(t, b):
        return pltpu.make_async_copy(
            g_sp.at[src_v.at[pl.ds(t * KB, KB)]], rows_v.at[b], gsem.at[b])

    def s_desc(t, b):
        return pltpu.make_async_copy(
            rows_v.at[b], acc_sh.at[dst_v.at[pl.ds(t * KB, KB)]], ssem.at[b])

    for b in range(NBUF):
        g_desc(b, b).start()

    def group(i, _):
        for b in range(NBUF):
            t = i * NBUF + b
            g_desc(t, b).wait()
            pltpu.async_copy(rows_v.at[b],
                             acc_sh.at[dst_v.at[pl.ds(t * KB, KB)]],
                             ssem.at[b], add=True)
        for b in range(NBUF):
            t = i * NBUF + b
            s_desc(t, b).wait()
            g_desc(t + NBUF, b).start()
        return 0
    lax.fori_loop(0, G - 1, group, 0)

    for b in range(NBUF):
        t = (G - 1) * NBUF + b
        g_desc(t, b).wait()
        pltpu.async_copy(rows_v.at[b],
                         acc_sh.at[dst_v.at[pl.ds(t * KB, KB)]],
                         ssem.at[b], add=True)
    for b in range(NBUF):
        s_desc((G - 1) * NBUF + b, b).wait()

    plsc.subcore_barrier()
    pltpu.sync_copy(acc_sh.at[sl], stage_v)
    sel = jnp.where(c == 1, 1.0, 0.0).astype(jnp.float32)

    def erow(i, _):
        stage_v[i, :] = dinv_v[i, :] * (stage_v[i, :] + sel * g2_v[i, :])
        return 0
    lax.fori_loop(0, RPT, erow, 0)
    pltpu.sync_copy(stage_v, out_hbm.at[c, sl])


def _tc1_body(degp_ref, x_ref, w1_ref, g1_ref, dinv_ref):
    deg = degp_ref[0] + degp_ref[1] + 1.0
    dinv = lax.rsqrt(deg)
    h = jnp.dot(x_ref[:], w1_ref[:], preferred_element_type=jnp.float32)
    g1_ref[:N, :] = h * dinv[:N, :]
    g1_ref[N:, :] = jnp.zeros((NPAD - N, HID), jnp.float32)
    dinv_ref[:] = dinv


def _tc3_body(zp_ref, w2_ref, b2_ref, out_ref):
    z = zp_ref[0, :N, :] + zp_ref[1, :N, :]
    out_ref[:] = (
        jnp.dot(z, w2_ref[:], preferred_element_type=jnp.float32) + b2_ref[:]
    )


def kernel(x, edge_index, W1, b1, W2, b2):
    b1r = b1.reshape(1, HID)
    b2r = b2.reshape(1, OUT_CH)

    deg_parts = _deg_kernel(edge_index)

    g1, dinv16 = pl.pallas_call(
        _tc1_body,
        out_shape=(
            jax.ShapeDtypeStruct((NPAD, HID), jnp.float32),
            jax.ShapeDtypeStruct((NPAD, HID), jnp.float32),
        ),
    )(deg_parts, x, W1)

    acc1 = _agg_kernel(g1, edge_index)

    zp = _agg2_kernel(acc1, g1, dinv16, b1r, edge_index)

    out = pl.pallas_call(
        _tc3_body,
        out_shape=jax.ShapeDtypeStruct((N, OUT_CH), jnp.float32),
    )(zp, W2, b2r)

    return out


# speedup vs baseline: 1.7338x; 1.1077x over previous
import functools

import jax
import jax.numpy as jnp
from jax import lax
from jax.experimental import pallas as pl
from jax.experimental.pallas import tpu as pltpu
from jax.experimental.pallas import tpu_sc as plsc

N = 10000
E = 320000
IN_CH = 128
HID = 16
OUT_CH = 128

NC = 2
NS = 16
NW = NC * NS
L = 16

EPW = E // NW
NBUF = 5
KB = 512
SCH = 20
EPW_P = SCH * KB
G = SCH // NBUF
RPT = 632
NPAD = NS * RPT

_MESH = plsc.VectorSubcoreMesh(core_axis_name="c", subcore_axis_name="s")
_SC_PARAMS = pltpu.CompilerParams(use_tc_tiling_on_sc=False)


def _fill_tail(idx_v):
    for i in range((EPW_P - EPW) // L):
        idx_v[pl.ds(EPW + i * L, L)] = jnp.full((L,), N, jnp.int32)


def _zero_slice_and_barrier(stage_v, acc_sh, s):
    def zfill(i, _):
        for u in range(4):
            stage_v[i * 4 + u, :] = jnp.zeros((L,), jnp.float32)
        return 0
    lax.fori_loop(0, RPT // 4, zfill, 0)
    pltpu.sync_copy(stage_v, acc_sh.at[pl.ds(s * RPT, RPT)])
    plsc.subcore_barrier()


def _writeback(stage_v, acc_sh, out_hbm, c, s):
    plsc.subcore_barrier()
    pltpu.sync_copy(acc_sh.at[pl.ds(s * RPT, RPT)],
                    out_hbm.at[c, pl.ds(s * RPT, RPT)])


@functools.partial(
    pl.kernel,
    out_type=jax.ShapeDtypeStruct((NC, NPAD, L), jnp.float32),
    mesh=_MESH,
    scratch_types=[
        pltpu.VMEM((EPW_P,), jnp.int32),
        pltpu.VMEM((KB, L), jnp.float32),
        pltpu.VMEM((RPT, L), jnp.float32),
        pltpu.VMEM_SHARED((NPAD, L), jnp.float32),
        pltpu.SemaphoreType.DMA,
    ],
    compiler_params=_SC_PARAMS,
)
def _deg_kernel(ei_hbm, out_hbm, dst_v, ones_v, stage_v, acc_sh, sem):
    c = lax.axis_index("c")
    s = lax.axis_index("s")
    w = c * NS + s

    def ofill(i, _):
        for u in range(4):
            ones_v[i * 4 + u, :] = jnp.full((L,), 1.0, jnp.float32)
        return 0
    lax.fori_loop(0, KB // 4, ofill, 0)
    _zero_slice_and_barrier(stage_v, acc_sh, s)

    pltpu.sync_copy(ei_hbm.at[1, pl.ds(w * EPW, EPW)],
                    dst_v.at[pl.ds(0, EPW)])
    _fill_tail(dst_v)

    def fire(j, _):
        pltpu.async_copy(ones_v, acc_sh.at[dst_v.at[pl.ds(j * KB, KB)]],
                         sem, add=True)
        return 0
    lax.fori_loop(0, SCH, fire, 0)

    def drain(j, _):
        pltpu.make_async_copy(ones_v,
                              acc_sh.at[dst_v.at[pl.ds(j * KB, KB)]],
                              sem).wait()
        return 0
    lax.fori_loop(0, SCH, drain, 0)

    _writeback(stage_v, acc_sh, out_hbm, c, s)


@functools.partial(
    pl.kernel,
    out_type=jax.ShapeDtypeStruct((NC, NPAD, L), jnp.float32),
    mesh=_MESH,
    scratch_types=[
        pltpu.VMEM((EPW_P,), jnp.int32),
        pltpu.VMEM((EPW_P,), jnp.int32),
        pltpu.VMEM((NBUF, KB, L), jnp.float32),
        pltpu.VMEM((RPT, L), jnp.float32),
        pltpu.VMEM_SHARED((NPAD, L), jnp.float32),
        pltpu.VMEM_SHARED((NPAD, L), jnp.float32),
        pltpu.SemaphoreType.DMA((NBUF,)),
        pltpu.SemaphoreType.DMA((NBUF,)),
    ],
    compiler_params=_SC_PARAMS,
)
def _agg_kernel(g_hbm, ei_hbm, out_hbm,
                src_v, dst_v, rows_v, stage_v, acc_sh, g_sp, gsem, ssem):
    c = lax.axis_index("c")
    s = lax.axis_index("s")
    w = c * NS + s

    pltpu.sync_copy(g_hbm.at[pl.ds(s * RPT, RPT)],
                    g_sp.at[pl.ds(s * RPT, RPT)])
    _zero_slice_and_barrier(stage_v, acc_sh, s)

    pltpu.sync_copy(ei_hbm.at[0, pl.ds(w * EPW, EPW)],
                    src_v.at[pl.ds(0, EPW)])
    pltpu.sync_copy(ei_hbm.at[1, pl.ds(w * EPW, EPW)],
                    dst_v.at[pl.ds(0, EPW)])
    _fill_tail(src_v)
    _fill_tail(dst_v)

    def g_desc(t, b):
        return pltpu.make_async_copy(
            g_sp.at[src_v.at[pl.ds(t * KB, KB)]], rows_v.at[b], gsem.at[b])

    def s_desc(t, b):
        return pltpu.make_async_copy(
            rows_v.at[b], acc_sh.at[dst_v.at[pl.ds(t * KB, KB)]], ssem.at[b])

    for b in range(NBUF):
        g_desc(b, b).start()

    def group(i, _):
        for b in range(NBUF):
            t = i * NBUF + b
            g_desc(t, b).wait()
            pltpu.async_copy(rows_v.at[b],
                             acc_sh.at[dst_v.at[pl.ds(t * KB, KB)]],
                             ssem.at[b], add=True)
        for b in range(NBUF):
            t = i * NBUF + b
            s_desc(t, b).wait()
            g_desc(t + NBUF, b).start()
        return 0
    lax.fori_loop(0, G - 1, group, 0)

    for b in range(NBUF):
        t = (G - 1) * NBUF + b
        g_desc(t, b).wait()
        pltpu.async_copy(rows_v.at[b],
                         acc_sh.at[dst_v.at[pl.ds(t * KB, KB)]],
                         ssem.at[b], add=True)
    for b in range(NBUF):
        s_desc((G - 1) * NBUF + b, b).wait()

    _writeback(stage_v, acc_sh, out_hbm, c, s)


@functools.partial(
    pl.kernel,
    out_type=jax.ShapeDtypeStruct((NC, NPAD, L), jnp.float32),
    mesh=_MESH,
    scratch_types=[
        pltpu.VMEM((EPW_P,), jnp.int32),
        pltpu.VMEM((EPW_P,), jnp.int32),
        pltpu.VMEM((NBUF, KB, L), jnp.float32),
        pltpu.VMEM((RPT, L), jnp.float32),
        pltpu.VMEM((RPT, L), jnp.float32),
        pltpu.VMEM((RPT, L), jnp.float32),
        pltpu.VMEM((RPT, L), jnp.float32),
        pltpu.VMEM((1, L), jnp.float32),
        pltpu.VMEM_SHARED((NPAD, L), jnp.float32),
        pltpu.VMEM_SHARED((NPAD, L), jnp.float32),
        pltpu.SemaphoreType.DMA((NBUF,)),
        pltpu.SemaphoreType.DMA((NBUF,)),
    ],
    compiler_params=_SC_PARAMS,
)
def _agg2_kernel(accp_hbm, g1_hbm, dinv_hbm, b1_hbm, ei_hbm, out_hbm,
                 src_v, dst_v, rows_v, stage_v, tmp_v, dinv_v, g2_v, b1_v,
                 acc_sh, g_sp, gsem, ssem):
    c = lax.axis_index("c")
    s = lax.axis_index("s")
    w = c * NS + s
    sl = pl.ds(s * RPT, RPT)

    pltpu.sync_copy(accp_hbm.at[0, sl], stage_v)
    pltpu.sync_copy(accp_hbm.at[1, sl], tmp_v)
    pltpu.sync_copy(g1_hbm.at[sl], g2_v)
    pltpu.sync_copy(dinv_hbm.at[sl], dinv_v)
    pltpu.sync_copy(b1_hbm, b1_v)

    def prow(i, _):
        for u in range(4):
            r = i * 4 + u
            acc = stage_v[r, :] + tmp_v[r, :] + g2_v[r, :]
            z = dinv_v[r, :] * acc + b1_v[0, :]
            g2_v[r, :] = jnp.maximum(z, 0.0) * dinv_v[r, :]
        return 0
    lax.fori_loop(0, RPT // 4, prow, 0)
    pltpu.sync_copy(g2_v, g_sp.at[sl])
    _zero_slice_and_barrier(stage_v, acc_sh, s)

    pltpu.sync_copy(ei_hbm.at[0, pl.ds(w * EPW, EPW)],
                    src_v.at[pl.ds(0, EPW)])
    pltpu.sync_copy(ei_hbm.at[1, pl.ds(w * EPW, EPW)],
                    dst_v.at[pl.ds(0, EPW)])
    _fill_tail(src_v)
    _fill_tail(dst_v)

    def g_desc(t, b):
        return pltpu.make_async_copy(
            g_sp.at[src_v.at[pl.ds(t * KB, KB)]], rows_v.at[b], gsem.at[b])

    def s_desc(t, b):
        return pltpu.make_async_copy(
            rows_v.at[b], acc_sh.at[dst_v.at[pl.ds(t * KB, KB)]], ssem.at[b])

    for b in range(NBUF):
        g_desc(b, b).start()

    def group(i, _):
        for b in range(NBUF):
            t = i * NBUF + b
            g_desc(t, b).wait()
            pltpu.async_copy(rows_v.at[b],
                             acc_sh.at[dst_v.at[pl.ds(t * KB, KB)]],
                             ssem.at[b], add=True)
        for b in range(NBUF):
            t = i * NBUF + b
            s_desc(t, b).wait()
            g_desc(t + NBUF, b).start()
        return 0
    lax.fori_loop(0, G - 1, group, 0)

    for b in range(NBUF):
        t = (G - 1) * NBUF + b
        g_desc(t, b).wait()
        pltpu.async_copy(rows_v.at[b],
                         acc_sh.at[dst_v.at[pl.ds(t * KB, KB)]],
                         ssem.at[b], add=True)
    for b in range(NBUF):
        s_desc((G - 1) * NBUF + b, b).wait()

    plsc.subcore_barrier()
    pltpu.sync_copy(acc_sh.at[sl], stage_v)
    sel = jnp.where(c == 1, 1.0, 0.0).astype(jnp.float32)

    def erow(i, _):
        for u in range(4):
            r = i * 4 + u
            stage_v[r, :] = dinv_v[r, :] * (stage_v[r, :] + sel * g2_v[r, :])
        return 0
    lax.fori_loop(0, RPT // 4, erow, 0)
    pltpu.sync_copy(stage_v, out_hbm.at[c, sl])


def _tc1_body(degp_ref, x_ref, w1_ref, g1_ref, dinv_ref):
    deg = degp_ref[0] + degp_ref[1] + 1.0
    dinv = lax.rsqrt(deg)
    h = jnp.dot(x_ref[:], w1_ref[:], preferred_element_type=jnp.float32)
    g1_ref[:N, :] = h * dinv[:N, :]
    g1_ref[N:, :] = jnp.zeros((NPAD - N, HID), jnp.float32)
    dinv_ref[:] = dinv


def _tc3_body(zp_ref, w2_ref, b2_ref, out_ref):
    z = zp_ref[0, :N, :] + zp_ref[1, :N, :]
    out_ref[:] = (
        jnp.dot(z, w2_ref[:], preferred_element_type=jnp.float32) + b2_ref[:]
    )


def kernel(x, edge_index, W1, b1, W2, b2):
    b1r = b1.reshape(1, HID)
    b2r = b2.reshape(1, OUT_CH)

    deg_parts = _deg_kernel(edge_index)

    g1, dinv16 = pl.pallas_call(
        _tc1_body,
        out_shape=(
            jax.ShapeDtypeStruct((NPAD, HID), jnp.float32),
            jax.ShapeDtypeStruct((NPAD, HID), jnp.float32),
        ),
    )(deg_parts, x, W1)

    acc1 = _agg_kernel(g1, edge_index)

    zp = _agg2_kernel(acc1, g1, dinv16, b1r, edge_index)

    out = pl.pallas_call(
        _tc3_body,
        out_shape=jax.ShapeDtypeStruct((N, OUT_CH), jnp.float32),
    )(zp, W2, b2r)

    return out


# speedup vs baseline: 1.7392x; 1.0031x over previous
import functools

import jax
import jax.numpy as jnp
from jax import lax
from jax.experimental import pallas as pl
from jax.experimental.pallas import tpu as pltpu
from jax.experimental.pallas import tpu_sc as plsc

N = 10000
E = 320000
IN_CH = 128
HID = 16
OUT_CH = 128

NC = 2
NS = 16
NW = NC * NS
L = 16

EPW = E // NW
NBUF = 5
KB = 512
SCH = 20
EPW_P = SCH * KB
G = SCH // NBUF
RPT = 632
NPAD = NS * RPT

_MESH = plsc.VectorSubcoreMesh(core_axis_name="c", subcore_axis_name="s")
_SC_PARAMS = pltpu.CompilerParams(use_tc_tiling_on_sc=False)


def _fill_tail(idx_v):
    for i in range((EPW_P - EPW) // L):
        idx_v[pl.ds(EPW + i * L, L)] = jnp.full((L,), N, jnp.int32)


def _zero_slice_and_barrier(stage_v, acc_sh, s):
    def zfill(i, _):
        for u in range(8):
            stage_v[i * 8 + u, :] = jnp.zeros((L,), jnp.float32)
        return 0
    lax.fori_loop(0, RPT // 8, zfill, 0)
    pltpu.sync_copy(stage_v, acc_sh.at[pl.ds(s * RPT, RPT)])
    plsc.subcore_barrier()


def _writeback(stage_v, acc_sh, out_hbm, c, s):
    plsc.subcore_barrier()
    pltpu.sync_copy(acc_sh.at[pl.ds(s * RPT, RPT)],
                    out_hbm.at[c, pl.ds(s * RPT, RPT)])


@functools.partial(
    pl.kernel,
    out_type=jax.ShapeDtypeStruct((NC, NPAD, L), jnp.float32),
    mesh=_MESH,
    scratch_types=[
        pltpu.VMEM((EPW_P,), jnp.int32),
        pltpu.VMEM((KB, L), jnp.float32),
        pltpu.VMEM((RPT, L), jnp.float32),
        pltpu.VMEM_SHARED((NPAD, L), jnp.float32),
        pltpu.SemaphoreType.DMA,
    ],
    compiler_params=_SC_PARAMS,
)
def _deg_kernel(ei_hbm, out_hbm, dst_v, ones_v, stage_v, acc_sh, sem):
    c = lax.axis_index("c")
    s = lax.axis_index("s")
    w = c * NS + s

    def ofill(i, _):
        for u in range(8):
            ones_v[i * 8 + u, :] = jnp.full((L,), 1.0, jnp.float32)
        return 0
    lax.fori_loop(0, KB // 8, ofill, 0)
    _zero_slice_and_barrier(stage_v, acc_sh, s)

    pltpu.sync_copy(ei_hbm.at[1, pl.ds(w * EPW, EPW)],
                    dst_v.at[pl.ds(0, EPW)])
    _fill_tail(dst_v)

    def fire(j, _):
        pltpu.async_copy(ones_v, acc_sh.at[dst_v.at[pl.ds(j * KB, KB)]],
                         sem, add=True)
        return 0
    lax.fori_loop(0, SCH, fire, 0)

    def drain(j, _):
        pltpu.make_async_copy(ones_v,
                              acc_sh.at[dst_v.at[pl.ds(j * KB, KB)]],
                              sem).wait()
        return 0
    lax.fori_loop(0, SCH, drain, 0)

    _writeback(stage_v, acc_sh, out_hbm, c, s)


@functools.partial(
    pl.kernel,
    out_type=jax.ShapeDtypeStruct((NC, NPAD, L), jnp.float32),
    mesh=_MESH,
    scratch_types=[
        pltpu.VMEM((EPW_P,), jnp.int32),
        pltpu.VMEM((EPW_P,), jnp.int32),
        pltpu.VMEM((NBUF, KB, L), jnp.float32),
        pltpu.VMEM((RPT, L), jnp.float32),
        pltpu.VMEM_SHARED((NPAD, L), jnp.float32),
        pltpu.VMEM_SHARED((NPAD, L), jnp.float32),
        pltpu.SemaphoreType.DMA((NBUF,)),
        pltpu.SemaphoreType.DMA((NBUF,)),
    ],
    compiler_params=_SC_PARAMS,
)
def _agg_kernel(g_hbm, ei_hbm, out_hbm,
                src_v, dst_v, rows_v, stage_v, acc_sh, g_sp, gsem, ssem):
    c = lax.axis_index("c")
    s = lax.axis_index("s")
    w = c * NS + s

    pltpu.sync_copy(g_hbm.at[pl.ds(s * RPT, RPT)],
                    g_sp.at[pl.ds(s * RPT, RPT)])
    _zero_slice_and_barrier(stage_v, acc_sh, s)

    pltpu.sync_copy(ei_hbm.at[0, pl.ds(w * EPW, EPW)],
                    src_v.at[pl.ds(0, EPW)])
    pltpu.sync_copy(ei_hbm.at[1, pl.ds(w * EPW, EPW)],
                    dst_v.at[pl.ds(0, EPW)])
    _fill_tail(src_v)
    _fill_tail(dst_v)

    def g_desc(t, b):
        return pltpu.make_async_copy(
            g_sp.at[src_v.at[pl.ds(t * KB, KB)]], rows_v.at[b], gsem.at[b])

    def s_desc(t, b):
        return pltpu.make_async_copy(
            rows_v.at[b], acc_sh.at[dst_v.at[pl.ds(t * KB, KB)]], ssem.at[b])

    for b in range(NBUF):
        g_desc(b, b).start()

    def group(i, _):
        for b in range(NBUF):
            t = i * NBUF + b
            g_desc(t, b).wait()
            pltpu.async_copy(rows_v.at[b],
                             acc_sh.at[dst_v.at[pl.ds(t * KB, KB)]],
                             ssem.at[b], add=True)
        for b in range(NBUF):
            t = i * NBUF + b
            s_desc(t, b).wait()
            g_desc(t + NBUF, b).start()
        return 0
    lax.fori_loop(0, G - 1, group, 0)

    for b in range(NBUF):
        t = (G - 1) * NBUF + b
        g_desc(t, b).wait()
        pltpu.async_copy(rows_v.at[b],
                         acc_sh.at[dst_v.at[pl.ds(t * KB, KB)]],
                         ssem.at[b], add=True)
    for b in range(NBUF):
        s_desc((G - 1) * NBUF + b, b).wait()

    _writeback(stage_v, acc_sh, out_hbm, c, s)


@functools.partial(
    pl.kernel,
    out_type=jax.ShapeDtypeStruct((NC, NPAD, L), jnp.float32),
    mesh=_MESH,
    scratch_types=[
        pltpu.VMEM((EPW_P,), jnp.int32),
        pltpu.VMEM((EPW_P,), jnp.int32),
        pltpu.VMEM((NBUF, KB, L), jnp.float32),
        pltpu.VMEM((RPT, L), jnp.float32),
        pltpu.VMEM((RPT, L), jnp.float32),
        pltpu.VMEM((RPT, L), jnp.float32),
        pltpu.VMEM((RPT, L), jnp.float32),
        pltpu.VMEM((1, L), jnp.float32),
        pltpu.VMEM_SHARED((NPAD, L), jnp.float32),
        pltpu.VMEM_SHARED((NPAD, L), jnp.float32),
        pltpu.SemaphoreType.DMA((NBUF,)),
        pltpu.SemaphoreType.DMA((NBUF,)),
    ],
    compiler_params=_SC_PARAMS,
)
def _agg2_kernel(accp_hbm, g1_hbm, dinv_hbm, b1_hbm, ei_hbm, out_hbm,
                 src_v, dst_v, rows_v, stage_v, tmp_v, dinv_v, g2_v, b1_v,
                 acc_sh, g_sp, gsem, ssem):
    c = lax.axis_index("c")
    s = lax.axis_index("s")
    w = c * NS + s
    sl = pl.ds(s * RPT, RPT)

    pltpu.sync_copy(accp_hbm.at[0, sl], stage_v)
    pltpu.sync_copy(accp_hbm.at[1, sl], tmp_v)
    pltpu.sync_copy(g1_hbm.at[sl], g2_v)
    pltpu.sync_copy(dinv_hbm.at[sl], dinv_v)
    pltpu.sync_copy(b1_hbm, b1_v)

    def prow(i, _):
        for u in range(8):
            r = i * 8 + u
            acc = stage_v[r, :] + tmp_v[r, :] + g2_v[r, :]
            z = dinv_v[r, :] * acc + b1_v[0, :]
            g2_v[r, :] = jnp.maximum(z, 0.0) * dinv_v[r, :]
        return 0
    lax.fori_loop(0, RPT // 8, prow, 0)
    pltpu.sync_copy(g2_v, g_sp.at[sl])
    _zero_slice_and_barrier(stage_v, acc_sh, s)

    pltpu.sync_copy(ei_hbm.at[0, pl.ds(w * EPW, EPW)],
                    src_v.at[pl.ds(0, EPW)])
    pltpu.sync_copy(ei_hbm.at[1, pl.ds(w * EPW, EPW)],
                    dst_v.at[pl.ds(0, EPW)])
    _fill_tail(src_v)
    _fill_tail(dst_v)

    def g_desc(t, b):
        return pltpu.make_async_copy(
            g_sp.at[src_v.at[pl.ds(t * KB, KB)]], rows_v.at[b], gsem.at[b])

    def s_desc(t, b):
        return pltpu.make_async_copy(
            rows_v.at[b], acc_sh.at[dst_v.at[pl.ds(t * KB, KB)]], ssem.at[b])

    for b in range(NBUF):
        g_desc(b, b).start()

    def group(i, _):
        for b in range(NBUF):
            t = i * NBUF + b
            g_desc(t, b).wait()
            pltpu.async_copy(rows_v.at[b],
                             acc_sh.at[dst_v.at[pl.ds(t * KB, KB)]],
                             ssem.at[b], add=True)
        for b in range(NBUF):
            t = i * NBUF + b
            s_desc(t, b).wait()
            g_desc(t + NBUF, b).start()
        return 0
    lax.fori_loop(0, G - 1, group, 0)

    for b in range(NBUF):
        t = (G - 1) * NBUF + b
        g_desc(t, b).wait()
        pltpu.async_copy(rows_v.at[b],
                         acc_sh.at[dst_v.at[pl.ds(t * KB, KB)]],
                         ssem.at[b], add=True)
    for b in range(NBUF):
        s_desc((G - 1) * NBUF + b, b).wait()

    plsc.subcore_barrier()
    pltpu.sync_copy(acc_sh.at[sl], stage_v)
    sel = jnp.where(c == 1, 1.0, 0.0).astype(jnp.float32)

    def erow(i, _):
        for u in range(8):
            r = i * 8 + u
            stage_v[r, :] = dinv_v[r, :] * (stage_v[r, :] + sel * g2_v[r, :])
        return 0
    lax.fori_loop(0, RPT // 8, erow, 0)
    pltpu.sync_copy(stage_v, out_hbm.at[c, sl])


def _tc1_body(degp_ref, x_ref, w1_ref, g1_ref, dinv_ref):
    deg = degp_ref[0] + degp_ref[1] + 1.0
    dinv = lax.rsqrt(deg)
    h = jnp.dot(x_ref[:], w1_ref[:], preferred_element_type=jnp.float32)
    g1_ref[:N, :] = h * dinv[:N, :]
    g1_ref[N:, :] = jnp.zeros((NPAD - N, HID), jnp.float32)
    dinv_ref[:] = dinv


def _tc3_body(zp_ref, w2_ref, b2_ref, out_ref):
    z = zp_ref[0, :N, :] + zp_ref[1, :N, :]
    out_ref[:] = (
        jnp.dot(z, w2_ref[:], preferred_element_type=jnp.float32) + b2_ref[:]
    )


def kernel(x, edge_index, W1, b1, W2, b2):
    b1r = b1.reshape(1, HID)
    b2r = b2.reshape(1, OUT_CH)

    deg_parts = _deg_kernel(edge_index)

    g1, dinv16 = pl.pallas_call(
        _tc1_body,
        out_shape=(
            jax.ShapeDtypeStruct((NPAD, HID), jnp.float32),
            jax.ShapeDtypeStruct((NPAD, HID), jnp.float32),
        ),
    )(deg_parts, x, W1)

    acc1 = _agg_kernel(g1, edge_index)

    zp = _agg2_kernel(acc1, g1, dinv16, b1r, edge_index)

    out = pl.pallas_call(
        _tc3_body,
        out_shape=jax.ShapeDtypeStruct((N, OUT_CH), jnp.float32),
    )(zp, W2, b2r)

    return out
